# baseline (speedup 1.0000x reference)
import functools

import jax
import jax.numpy as jnp
from jax import lax
from jax.experimental import pallas as pl
from jax.experimental.pallas import tpu as pltpu
from jax.experimental.pallas import tpu_sc as plsc

N_NODES = 10000
D = 128
N_EDGES = 320000

N_SUBCORES = 16
CHUNK = 80
NBUF = 4
SUPER = 4
NIDX = 4
N_SUPERS = 64
N_QUADS = N_SUPERS // 4
CHUNKS_PER_TILE = SUPER * N_SUPERS
EDGES_PER_TILE = CHUNKS_PER_TILE * CHUNK
E_PAD = EDGES_PER_TILE * N_SUBCORES
ACC_ROWS = N_NODES + 8
ZERO_ROWS_MAIN = 632
ZERO_ROWS_LAST = ACC_ROWS - 15 * ZERO_ROWS_MAIN
OUT_ROWS_MAIN = 632
OUT_ROWS_LAST = N_NODES - 15 * OUT_ROWS_MAIN

TC_BLOCK = 1000


def _pre_body(h_ref, c_ref, wf_ref, bf_ref, out_ref):
    z = jnp.dot(h_ref[...], wf_ref[...], preferred_element_type=jnp.float32)
    out_ref[...] = jax.nn.sigmoid(z + bf_ref[...]) * c_ref[...]


def _pre(h, c, Wf, bf2d):
    return pl.pallas_call(
        _pre_body,
        grid=(N_NODES // TC_BLOCK,),
        in_specs=[
            pl.BlockSpec((TC_BLOCK, D), lambda i: (i, 0)),
            pl.BlockSpec((TC_BLOCK, D), lambda i: (i, 0)),
            pl.BlockSpec((D, D), lambda i: (0, 0)),
            pl.BlockSpec((1, D), lambda i: (0, 0)),
        ],
        out_specs=pl.BlockSpec((TC_BLOCK, D), lambda i: (i, 0)),
        out_shape=jax.ShapeDtypeStruct((N_NODES, D), jnp.float32),
    )(h, c, Wf, bf2d)


def _sc_body(h_hbm, prod_hbm, src_hbm, dst_hbm, zero_hbm,
             hs_out, fc_out,
             si, di, bufs, acc, gsem, ssem, isem):
    cid = lax.axis_index("c")
    sid = lax.axis_index("s")

    @pl.when(sid < 15)
    def _():
        pltpu.sync_copy(zero_hbm.at[pl.ds(sid * ZERO_ROWS_MAIN, ZERO_ROWS_MAIN)],
                        acc.at[pl.ds(sid * ZERO_ROWS_MAIN, ZERO_ROWS_MAIN)])

    @pl.when(sid == 15)
    def _():
        pltpu.sync_copy(zero_hbm.at[pl.ds(15 * ZERO_ROWS_MAIN, ZERO_ROWS_LAST)],
                        acc.at[pl.ds(15 * ZERO_ROWS_MAIN, ZERO_ROWS_LAST)])

    plsc.subcore_barrier()

    def run(table, out):
        def stage(s, par):
            pltpu.async_copy(src_hbm.at[sid, s], si[par], isem[par])
            pltpu.async_copy(dst_hbm.at[sid, s], di[par], isem[par])

        def wait_idx(par):
            pltpu.make_async_copy(src_hbm.at[sid, 0], si[par], isem[par]).wait()
            pltpu.make_async_copy(src_hbm.at[sid, 0], di[par], isem[par]).wait()

        def start_gather(row, par, b):
            pltpu.async_copy(table.at[si[par].at[row]], bufs[b], gsem[b])

        def wait_gather(b):
            pltpu.make_async_copy(table.at[si[0].at[0]], bufs[b], gsem[b]).wait()

        def start_scatter(row, par, b):
            pltpu.async_copy(bufs[b], acc.at[di[par].at[row]], ssem[b], add=True)

        def wait_scatter(b):
            pltpu.make_async_copy(bufs[b], acc.at[di[0].at[0]], ssem[b]).wait()

        stage(0, 0)
        wait_idx(0)
        start_gather(0, 0, 0)
        start_gather(1, 0, 1)

        def quad(u, carry):
            s_base = u * 4
            for off in range(16):
                if off % 4 == 0:
                    sp = off // 4
                    tgt = (sp + 1) % 4
                    if sp == 3:
                        @pl.when(u < N_QUADS - 1)
                        def _():
                            stage(s_base + sp + 1, tgt)
                    else:
                        stage(s_base + sp + 1, tgt)

                g_par = ((off + 2) // 4) % 4
                g_row = (off + 2) % 4
                gb = (off + 2) % 4

                def issue(par=g_par, row=g_row, b=gb, idx_wait=((off + 2) % 4 == 0)):
                    if idx_wait:
                        wait_idx(par)
                    wait_scatter(b)
                    start_gather(row, par, b)

                if off in (0, 1):
                    @pl.when(u > 0)
                    def _():
                        issue()

                    @pl.when(u == 0)
                    def _(b=gb, row=g_row, par=g_par):
                        start_gather(row, par, b)
                elif off in (14, 15):
                    @pl.when(u < N_QUADS - 1)
                    def _():
                        issue()
                else:
                    issue()

                kb = off % 4
                wait_gather(kb)
                start_scatter(off % 4, (off // 4) % 4, kb)
            return carry

        lax.fori_loop(0, N_QUADS, quad, 0)
        for b in range(NBUF):
            wait_scatter(b)

        plsc.subcore_barrier()

        @pl.when(sid < 15)
        def _():
            pltpu.sync_copy(acc.at[pl.ds(sid * OUT_ROWS_MAIN, OUT_ROWS_MAIN)],
                            out.at[pl.ds(sid * OUT_ROWS_MAIN, OUT_ROWS_MAIN)])

        @pl.when(sid == 15)
        def _():
            pltpu.sync_copy(acc.at[pl.ds(15 * OUT_ROWS_MAIN, OUT_ROWS_LAST)],
                            out.at[pl.ds(15 * OUT_ROWS_MAIN, OUT_ROWS_LAST)])

    @pl.when(cid == 0)
    def _():
        run(h_hbm, hs_out)

    @pl.when(cid == 1)
    def _():
        run(prod_hbm, fc_out)


@functools.partial(
    pl.kernel,
    out_type=[
        jax.ShapeDtypeStruct((N_NODES, D), jnp.float32),
        jax.ShapeDtypeStruct((N_NODES, D), jnp.float32),
    ],
    mesh=plsc.VectorSubcoreMesh(core_axis_name="c", subcore_axis_name="s"),
    scratch_types=(
        [pltpu.VMEM((SUPER, CHUNK), jnp.int32) for _ in range(2 * NIDX)]
        + [pltpu.VMEM((CHUNK, D), jnp.float32) for _ in range(NBUF)]
        + [pltpu.VMEM_SHARED((ACC_ROWS, D), jnp.float32)]
        + [pltpu.SemaphoreType.DMA for _ in range(NBUF + NBUF + NIDX)]
    ),
)
def _sc_segsum(h_hbm, prod_hbm, src_hbm, dst_hbm, zero_hbm, hs_out, fc_out,
               si0, si1, si2, si3, di0, di1, di2, di3,
               b0, b1, b2, b3, acc,
               gs0, gs1, gs2, gs3, ss0, ss1, ss2, ss3, is0, is1, is2, is3):
    _sc_body(h_hbm, prod_hbm, src_hbm, dst_hbm, zero_hbm, hs_out, fc_out,
             (si0, si1, si2, si3), (di0, di1, di2, di3),
             (b0, b1, b2, b3), acc,
             (gs0, gs1, gs2, gs3), (ss0, ss1, ss2, ss3),
             (is0, is1, is2, is3))


def _post_body(hs_ref, fc_ref, wg_ref, bg_ref, out_ref):
    gates = jnp.dot(hs_ref[...], wg_ref[...], preferred_element_type=jnp.float32)
    gates = gates + bg_ref[...]
    i = jax.nn.sigmoid(gates[:, :D])
    o = jax.nn.sigmoid(gates[:, D:2 * D])
    g = jnp.tanh(gates[:, 2 * D:])
    c_new = i * g + fc_ref[...]
    h_new = o * jnp.tanh(c_new)
    out_ref[:, 0, :] = h_new
    out_ref[:, 1, :] = c_new


def _post(hs_sum, fc_reduce, Wg, bg2d):
    return pl.pallas_call(
        _post_body,
        grid=(N_NODES // TC_BLOCK,),
        in_specs=[
            pl.BlockSpec((TC_BLOCK, D), lambda i: (i, 0)),
            pl.BlockSpec((TC_BLOCK, D), lambda i: (i, 0)),
            pl.BlockSpec((D, 3 * D), lambda i: (0, 0)),
            pl.BlockSpec((1, 3 * D), lambda i: (0, 0)),
        ],
        out_specs=pl.BlockSpec((TC_BLOCK, 2, D), lambda i: (i, 0, 0)),
        out_shape=jax.ShapeDtypeStruct((N_NODES, 2, D), jnp.float32),
    )(hs_sum, fc_reduce, Wg, bg2d)


def kernel(h, c, edge_index, Wg, bg, Wf, bf):
    src = edge_index[0]
    dst = edge_index[1]
    pad = E_PAD - N_EDGES
    src_p = jnp.concatenate([src, jnp.zeros((pad,), jnp.int32)])
    dst_p = jnp.concatenate([dst, jnp.full((pad,), N_NODES, jnp.int32)])
    src_p = src_p.reshape(N_SUBCORES, N_SUPERS, SUPER, CHUNK).astype(jnp.int32)
    dst_p = dst_p.reshape(N_SUBCORES, N_SUPERS, SUPER, CHUNK).astype(jnp.int32)
    zeros = jnp.zeros((ACC_ROWS, D), jnp.float32)

    prod = _pre(h, c, Wf, bf.reshape(1, D))
    hs_sum, fc_reduce = _sc_segsum(h, prod, src_p, dst_p, zeros)
    return _post(hs_sum, fc_reduce, Wg, bg.reshape(1, 3 * D))


# speedup vs baseline: 1.0557x; 1.0557x over previous
import functools

import jax
import jax.numpy as jnp
from jax import lax
from jax.experimental import pallas as pl
from jax.experimental.pallas import tpu as pltpu
from jax.experimental.pallas import tpu_sc as plsc

N_NODES = 10000
D = 128
N_EDGES = 320000

N_SUBCORES = 16
N_TILES = 32
CHUNK = 64
SUPER = 16
N_SUPERS = 10
CHUNKS_PER_TILE = SUPER * N_SUPERS
EDGES_PER_TILE = CHUNKS_PER_TILE * CHUNK
E_PAD = EDGES_PER_TILE * N_TILES
TW = 256
PACC_ROWS = 5008
PZERO_MAIN = 320
PZERO_LAST = PACC_ROWS - 15 * PZERO_MAIN
ROWS_PER_TILE_ACC = 632
ACC_ROWS = ROWS_PER_TILE_ACC * N_SUBCORES
OUT_ROWS_MAIN = 632
OUT_ROWS_LAST = N_NODES - 15 * OUT_ROWS_MAIN

TC_BLOCK = 1000


def _pre_body(h_ref, c_ref, wf_ref, bf_ref, out_ref):
    z = jnp.dot(h_ref[...], wf_ref[...], preferred_element_type=jnp.float32)
    out_ref[...] = jax.nn.sigmoid(z + bf_ref[...]) * c_ref[...]


def _pre(h, c, Wf, bf2d):
    return pl.pallas_call(
        _pre_body,
        grid=(N_NODES // TC_BLOCK,),
        in_specs=[
            pl.BlockSpec((TC_BLOCK, D), lambda i: (i, 0)),
            pl.BlockSpec((TC_BLOCK, D), lambda i: (i, 0)),
            pl.BlockSpec((D, D), lambda i: (0, 0)),
            pl.BlockSpec((1, D), lambda i: (0, 0)),
        ],
        out_specs=pl.BlockSpec((TC_BLOCK, D), lambda i: (i, 0)),
        out_shape=jax.ShapeDtypeStruct((N_NODES, D), jnp.float32),
    )(h, c, Wf, bf2d)


def _sc_body(tbl_hbm, src_hbm, dst_hbm, zero_hbm,
             o0, o1,
             src_v, dst_v, buf0, buf1, acc, sem0, sem1):
    cid = lax.axis_index("c")
    sid = lax.axis_index("s")
    widx = cid * N_SUBCORES + sid

    @pl.when(sid < 15)
    def _():
        pltpu.sync_copy(zero_hbm.at[pl.ds(sid * PZERO_MAIN, PZERO_MAIN)],
                        acc.at[pl.ds(sid * PZERO_MAIN, PZERO_MAIN)])

    @pl.when(sid == 15)
    def _():
        pltpu.sync_copy(zero_hbm.at[pl.ds(15 * PZERO_MAIN, PZERO_LAST)],
                        acc.at[pl.ds(15 * PZERO_MAIN, PZERO_LAST)])

    plsc.subcore_barrier()

    def run(table, out):
        def start(idx_row, buf, sem):
            pltpu.async_copy(table.at[idx_row], buf, sem)

        def wait(buf, sem):
            pltpu.make_async_copy(table.at[src_v.at[0]], buf, sem).wait()

        def scatter_add(idx_row, buf):
            pltpu.sync_copy(buf, acc.at[idx_row], add=True)

        def process_super(s, carry):
            pltpu.sync_copy(src_hbm.at[widx, pl.ds(s * SUPER, SUPER)], src_v)
            pltpu.sync_copy(dst_hbm.at[widx, pl.ds(s * SUPER, SUPER)], dst_v)
            start(src_v.at[0], buf0, sem0)

            def body(jj, carry2):
                p0 = jj * 2
                start(src_v.at[p0 + 1], buf1, sem1)
                wait(buf0, sem0)
                scatter_add(dst_v.at[p0], buf0)

                @pl.when(p0 + 2 < SUPER)
                def _():
                    start(src_v.at[p0 + 2], buf0, sem0)

                wait(buf1, sem1)
                scatter_add(dst_v.at[p0 + 1], buf1)
                return carry2

            lax.fori_loop(0, SUPER // 2, body, 0)
            return carry

        lax.fori_loop(0, N_SUPERS, process_super, 0)

        plsc.subcore_barrier()

        @pl.when(sid < 15)
        def _():
            pltpu.sync_copy(acc.at[pl.ds(sid * PZERO_MAIN, PZERO_MAIN)],
                            out.at[pl.ds(sid * PZERO_MAIN, PZERO_MAIN)])

        @pl.when(sid == 15)
        def _():
            pltpu.sync_copy(acc.at[pl.ds(15 * PZERO_MAIN, PZERO_LAST)],
                            out.at[pl.ds(15 * PZERO_MAIN, PZERO_LAST)])

    @pl.when(cid == 0)
    def _():
        run(tbl_hbm, o0)

    @pl.when(cid == 1)
    def _():
        run(tbl_hbm, o1)


@functools.partial(
    pl.kernel,
    out_type=[
        jax.ShapeDtypeStruct((PACC_ROWS, 2, D), jnp.float32),
        jax.ShapeDtypeStruct((PACC_ROWS, 2, D), jnp.float32),
    ],
    mesh=plsc.VectorSubcoreMesh(core_axis_name="c", subcore_axis_name="s"),
    scratch_types=[
        pltpu.VMEM((SUPER, CHUNK), jnp.int32),
        pltpu.VMEM((SUPER, CHUNK), jnp.int32),
        pltpu.VMEM((CHUNK, 2, D), jnp.float32),
        pltpu.VMEM((CHUNK, 2, D), jnp.float32),
        pltpu.VMEM_SHARED((PACC_ROWS, 2, D), jnp.float32),
        pltpu.SemaphoreType.DMA,
        pltpu.SemaphoreType.DMA,
    ],
)
def _sc_segsum(tbl_hbm, src_hbm, dst_hbm, zero_hbm, o0, o1,
               src_v, dst_v, buf0, buf1, acc, sem0, sem1):
    _sc_body(tbl_hbm, src_hbm, dst_hbm, zero_hbm, o0, o1,
             src_v, dst_v, buf0, buf1, acc, sem0, sem1)


def _post_body(hs_ref, fc_ref, wg_ref, bg_ref, out_ref):
    gates = jnp.dot(hs_ref[...], wg_ref[...], preferred_element_type=jnp.float32)
    gates = gates + bg_ref[...]
    i = jax.nn.sigmoid(gates[:, :D])
    o = jax.nn.sigmoid(gates[:, D:2 * D])
    g = jnp.tanh(gates[:, 2 * D:])
    c_new = i * g + fc_ref[...]
    h_new = o * jnp.tanh(c_new)
    out_ref[:, 0, :] = h_new
    out_ref[:, 1, :] = c_new


def _post(hs_sum, fc_reduce, Wg, bg2d):
    return pl.pallas_call(
        _post_body,
        grid=(N_NODES // TC_BLOCK,),
        in_specs=[
            pl.BlockSpec((TC_BLOCK, D), lambda i: (i, 0)),
            pl.BlockSpec((TC_BLOCK, D), lambda i: (i, 0)),
            pl.BlockSpec((D, 3 * D), lambda i: (0, 0)),
            pl.BlockSpec((1, 3 * D), lambda i: (0, 0)),
        ],
        out_specs=pl.BlockSpec((TC_BLOCK, 2, D), lambda i: (i, 0, 0)),
        out_shape=jax.ShapeDtypeStruct((N_NODES, 2, D), jnp.float32),
    )(hs_sum, fc_reduce, Wg, bg2d)


def kernel(h, c, edge_index, Wg, bg, Wf, bf):
    src = edge_index[0]
    dst = edge_index[1]
    pad = E_PAD - N_EDGES
    src_p = jnp.concatenate([src, jnp.zeros((pad,), jnp.int32)])
    dst_p = jnp.concatenate([dst // 2, jnp.full((pad,), 5000, jnp.int32)])
    src_p = src_p.reshape(N_TILES, CHUNKS_PER_TILE, CHUNK).astype(jnp.int32)
    dst_p = dst_p.reshape(N_TILES, CHUNKS_PER_TILE, CHUNK).astype(jnp.int32)
    zeros = jnp.zeros((PACC_ROWS, 2, D), jnp.float32)

    prod = _pre(h, c, Wf, bf.reshape(1, D))
    tbl = jnp.stack([h, prod], axis=1)
    o0, o1 = _sc_segsum(tbl, src_p, dst_p, zeros)
    both = o0 + o1
    hs_sum = jnp.concatenate([both, both], axis=0)[:N_NODES, 0, :]
    fc_reduce = jnp.concatenate([both, both], axis=0)[:N_NODES, 1, :]
    return _post(hs_sum, fc_reduce, Wg, bg.reshape(1, 3 * D))


# speedup vs baseline: 1.1475x; 1.0869x over previous
import functools

import jax
import jax.numpy as jnp
from jax import lax
from jax.experimental import pallas as pl
from jax.experimental.pallas import tpu as pltpu
from jax.experimental.pallas import tpu_sc as plsc

N_NODES = 10000
D = 128
N_EDGES = 320000

N_SUBCORES = 16
CHUNK = 128
SUPER = 16
N_SUPERS = 10
CHUNKS_PER_TILE = SUPER * N_SUPERS
EDGES_PER_TILE = CHUNKS_PER_TILE * CHUNK
E_PAD = EDGES_PER_TILE * N_SUBCORES
ROWS_PER_TILE_ACC = 632
ACC_ROWS = ROWS_PER_TILE_ACC * N_SUBCORES
OUT_ROWS_MAIN = 632
OUT_ROWS_LAST = N_NODES - 15 * OUT_ROWS_MAIN

TC_BLOCK = 1000


def _pre_body(h_ref, c_ref, wf_ref, bf_ref, out_ref):
    z = jnp.dot(h_ref[...], wf_ref[...], preferred_element_type=jnp.float32)
    out_ref[...] = jax.nn.sigmoid(z + bf_ref[...]) * c_ref[...]


def _pre(h, c, Wf, bf2d):
    return pl.pallas_call(
        _pre_body,
        grid=(N_NODES // TC_BLOCK,),
        in_specs=[
            pl.BlockSpec((TC_BLOCK, D), lambda i: (i, 0)),
            pl.BlockSpec((TC_BLOCK, D), lambda i: (i, 0)),
            pl.BlockSpec((D, D), lambda i: (0, 0)),
            pl.BlockSpec((1, D), lambda i: (0, 0)),
        ],
        out_specs=pl.BlockSpec((TC_BLOCK, D), lambda i: (i, 0)),
        out_shape=jax.ShapeDtypeStruct((N_NODES, D), jnp.float32),
    )(h, c, Wf, bf2d)


def _sc_body(h_hbm, prod_hbm, src_hbm, dst_hbm, zero_hbm,
             hs_out, fc_out,
             si0, si1, di0, di1, buf0, buf1, acc, sem0, sem1, isem):
    cid = lax.axis_index("c")
    sid = lax.axis_index("s")

    pltpu.sync_copy(zero_hbm.at[pl.ds(sid * ROWS_PER_TILE_ACC, ROWS_PER_TILE_ACC)],
                    acc.at[pl.ds(sid * ROWS_PER_TILE_ACC, ROWS_PER_TILE_ACC)])
    plsc.subcore_barrier()

    def run(table, out):
        si = (si0, si1)
        di = (di0, di1)
        bufs = (buf0, buf1)
        sems = (sem0, sem1)

        def stage(s, par):
            pltpu.async_copy(src_hbm.at[sid, pl.ds(s * SUPER, SUPER)],
                             si[par], isem)
            pltpu.async_copy(dst_hbm.at[sid, pl.ds(s * SUPER, SUPER)],
                             di[par], isem)

        def wait_idx(par):
            pltpu.make_async_copy(src_hbm.at[sid, pl.ds(0, SUPER)],
                                  si[par], isem).wait()
            pltpu.make_async_copy(src_hbm.at[sid, pl.ds(0, SUPER)],
                                  di[par], isem).wait()

        def start(par, row, b):
            pltpu.async_copy(table.at[si[par].at[row]], bufs[b], sems[b])

        def wait(b):
            pltpu.make_async_copy(table.at[si[0].at[0]], bufs[b], sems[b]).wait()

        def scatter_add(par, row, b):
            pltpu.sync_copy(bufs[b], acc.at[di[par].at[row]], add=True)

        stage(0, 0)
        wait_idx(0)
        stage(1, 1)
        start(0, 0, 0)
        start(0, 1, 1)

        def pair(t, carry):
            for half in range(2):
                s = 2 * t + half
                par = half

                if half == 0:
                    @pl.when(t > 0)
                    def _():
                        stage(s + 1, 1)
                else:
                    @pl.when(t < (N_SUPERS // 2) - 1)
                    def _():
                        stage(s + 1, 0)

                for p in range(SUPER):
                    b = p % 2
                    wait(b)
                    scatter_add(par, p, b)
                    if p + 2 < SUPER:
                        start(par, p + 2, b)
                    else:
                        nxt = 1 - par
                        if half == 0:
                            if p == SUPER - 2:
                                wait_idx(nxt)
                            start(nxt, p + 2 - SUPER, b)
                        else:
                            @pl.when(t < (N_SUPERS // 2) - 1)
                            def _(p=p, nxt=nxt, b=b):
                                if p == SUPER - 2:
                                    wait_idx(nxt)
                                start(nxt, p + 2 - SUPER, b)
            return carry

        lax.fori_loop(0, N_SUPERS // 2, pair, 0)

        plsc.subcore_barrier()

        @pl.when(sid < 15)
        def _():
            pltpu.sync_copy(acc.at[pl.ds(sid * OUT_ROWS_MAIN, OUT_ROWS_MAIN)],
                            out.at[pl.ds(sid * OUT_ROWS_MAIN, OUT_ROWS_MAIN)])

        @pl.when(sid == 15)
        def _():
            pltpu.sync_copy(acc.at[pl.ds(15 * OUT_ROWS_MAIN, OUT_ROWS_LAST)],
                            out.at[pl.ds(15 * OUT_ROWS_MAIN, OUT_ROWS_LAST)])

    @pl.when(cid == 0)
    def _():
        run(h_hbm, hs_out)

    @pl.when(cid == 1)
    def _():
        run(prod_hbm, fc_out)


@functools.partial(
    pl.kernel,
    out_type=[
        jax.ShapeDtypeStruct((N_NODES, D), jnp.float32),
        jax.ShapeDtypeStruct((N_NODES, D), jnp.float32),
    ],
    mesh=plsc.VectorSubcoreMesh(core_axis_name="c", subcore_axis_name="s"),
    scratch_types=[
        pltpu.VMEM((SUPER, CHUNK), jnp.int32),
        pltpu.VMEM((SUPER, CHUNK), jnp.int32),
        pltpu.VMEM((SUPER, CHUNK), jnp.int32),
        pltpu.VMEM((SUPER, CHUNK), jnp.int32),
        pltpu.VMEM((CHUNK, D), jnp.float32),
        pltpu.VMEM((CHUNK, D), jnp.float32),
        pltpu.VMEM_SHARED((ACC_ROWS, D), jnp.float32),
        pltpu.SemaphoreType.DMA,
        pltpu.SemaphoreType.DMA,
        pltpu.SemaphoreType.DMA,
    ],
)
def _sc_segsum(h_hbm, prod_hbm, src_hbm, dst_hbm, zero_hbm, hs_out, fc_out,
               si0, si1, di0, di1, buf0, buf1, acc, sem0, sem1, isem):
    _sc_body(h_hbm, prod_hbm, src_hbm, dst_hbm, zero_hbm, hs_out, fc_out,
             si0, si1, di0, di1, buf0, buf1, acc, sem0, sem1, isem)


def _post_body(hs_ref, fc_ref, wg_ref, bg_ref, out_ref):
    gates = jnp.dot(hs_ref[...], wg_ref[...], preferred_element_type=jnp.float32)
    gates = gates + bg_ref[...]
    i = jax.nn.sigmoid(gates[:, :D])
    o = jax.nn.sigmoid(gates[:, D:2 * D])
    g = jnp.tanh(gates[:, 2 * D:])
    c_new = i * g + fc_ref[...]
    h_new = o * jnp.tanh(c_new)
    out_ref[:, 0, :] = h_new
    out_ref[:, 1, :] = c_new


def _post(hs_sum, fc_reduce, Wg, bg2d):
    return pl.pallas_call(
        _post_body,
        grid=(N_NODES // TC_BLOCK,),
        in_specs=[
            pl.BlockSpec((TC_BLOCK, D), lambda i: (i, 0)),
            pl.BlockSpec((TC_BLOCK, D), lambda i: (i, 0)),
            pl.BlockSpec((D, 3 * D), lambda i: (0, 0)),
            pl.BlockSpec((1, 3 * D), lambda i: (0, 0)),
        ],
        out_specs=pl.BlockSpec((TC_BLOCK, 2, D), lambda i: (i, 0, 0)),
        out_shape=jax.ShapeDtypeStruct((N_NODES, 2, D), jnp.float32),
    )(hs_sum, fc_reduce, Wg, bg2d)


def kernel(h, c, edge_index, Wg, bg, Wf, bf):
    src = edge_index[0]
    dst = edge_index[1]
    pad = E_PAD - N_EDGES
    src_p = jnp.concatenate([src, jnp.zeros((pad,), jnp.int32)])
    dst_p = jnp.concatenate([dst, jnp.full((pad,), N_NODES, jnp.int32)])
    src_p = src_p.reshape(N_SUBCORES, CHUNKS_PER_TILE, CHUNK).astype(jnp.int32)
    dst_p = dst_p.reshape(N_SUBCORES, CHUNKS_PER_TILE, CHUNK).astype(jnp.int32)
    zeros = jnp.zeros((ACC_ROWS, D), jnp.float32)

    prod = _pre(h, c, Wf, bf.reshape(1, D))
    hs_sum, fc_reduce = _sc_segsum(h, prod, src_p, dst_p, zeros)
    return _post(hs_sum, fc_reduce, Wg, bg.reshape(1, 3 * D))
